# baseline (speedup 1.0000x reference)
import jax
import jax.numpy as jnp
from jax import lax
from jax.experimental import pallas as pl
from jax.experimental.pallas import tpu as pltpu
from jax.experimental.pallas import tpu_sc as plsc

_NW = 32
_W = 128


def _pack_tc(table):
    n_vocab, emb = table.shape
    rows = 1024
    blocks = -(-n_vocab // rows)

    def body(in_ref, out_ref):
        out_ref[:, 0:emb] = in_ref[0:rows // 2, :]
        out_ref[:, emb:2 * emb] = in_ref[rows // 2:rows, :]

    return pl.pallas_call(
        body,
        grid=(blocks,),
        in_specs=[pl.BlockSpec((rows, emb), lambda i: (i, 0))],
        out_specs=pl.BlockSpec((rows // 2, 2 * emb), lambda i: (i, 0)),
        out_shape=jax.ShapeDtypeStruct((blocks * (rows // 2), 2 * emb),
                                       jnp.float32),
    )(table)


def _embed_sc(t2, tok3, n_idx):
    n_rows = tok3.shape[0]
    rows_per_w = n_rows // _NW
    toks_per_w = n_idx // _NW
    chunks = toks_per_w // _W

    mesh = plsc.VectorSubcoreMesh(core_axis_name="core",
                                  subcore_axis_name="subcore")

    @pl.kernel(
        out_type=jax.ShapeDtypeStruct((n_idx, 64), jnp.float32),
        mesh=mesh,
        scratch_types=[
            pltpu.VMEM((rows_per_w, _W), jnp.int32),
            pltpu.VMEM((_W,), jnp.int32),
            pltpu.VMEM((_W,), jnp.int32),
            pltpu.VMEM((_W, 128), jnp.float32),
            pltpu.VMEM((_W, 128), jnp.float32),
            pltpu.VMEM((_W, 64), jnp.float32),
            pltpu.SemaphoreType.DMA,
            pltpu.SemaphoreType.DMA,
            pltpu.SemaphoreType.DMA,
        ],
    )
    def k(t2_hbm, tok_hbm, out_hbm, idx_all, i0, i1, g0, g1, fbuf,
          sem_i, s0, s1):
        wid = lax.axis_index("subcore") * 2 + lax.axis_index("core")

        pltpu.async_copy(
            tok_hbm.at[pl.ds(wid * rows_per_w, rows_per_w)], idx_all, sem_i
        ).wait()

        base = wid * toks_per_w

        def gather(c, ibuf, gbuf, sem):
            @pl.loop(0, _W // 16)
            def _(v):
                tok = idx_all[c, pl.ds(v * 16, 16)]
                ibuf[pl.ds(v * 16, 16)] = (
                    lax.shift_left(lax.shift_right_logical(tok, 10), 9)
                    | (tok & 511))

            return pltpu.async_copy(t2_hbm.at[ibuf], gbuf, sem)

        def flush(c, gbuf):
            for v in range(_W // 16):
                tok_vec = idx_all[c, pl.ds(v * 16, 16)]
                for j16 in range(16):
                    j = v * 16 + j16
                    off = (lax.shift_right_logical(tok_vec[j16], 9) & 1) * 64
                    for q in range(0, 64, 16):
                        fbuf[j, pl.ds(q, 16)] = gbuf[j, pl.ds(off + q, 16)]
            pltpu.sync_copy(fbuf, out_hbm.at[pl.ds(base + c * _W, _W)])

        gather(0, i0, g0, s0).wait()

        @pl.loop(0, chunks - 1)
        def _(c):
            even = c % 2 == 0

            @pl.when(even)
            def _():
                cp = gather(c + 1, i1, g1, s1)
                flush(c, g0)
                cp.wait()

            @pl.when(jnp.logical_not(even))
            def _():
                cp = gather(c + 1, i0, g0, s0)
                flush(c, g1)
                cp.wait()

        @pl.when((chunks - 1) % 2 == 0)
        def _():
            flush(chunks - 1, g0)

        @pl.when((chunks - 1) % 2 == 1)
        def _():
            flush(chunks - 1, g1)

    return k(t2, tok3)


def kernel(tokens, table):
    batch, hist = tokens.shape
    n_vocab, emb = table.shape
    n_idx = batch * hist
    t2 = _pack_tc(table)
    tok3 = tokens.reshape(n_idx // 128, 128)
    out = _embed_sc(t2, tok3, n_idx)
    return out.reshape(batch, hist, emb)


# speedup vs baseline: 1.2991x; 1.2991x over previous
import jax
import jax.numpy as jnp
from jax.experimental import pallas as pl
from jax.experimental.pallas import tpu as pltpu
from jax.experimental.pallas import tpu_sc as plsc

_WINDOW = 128


def _embed_sc(tokens_flat, table):
    n_idx = tokens_flat.shape[0]
    emb = table.shape[1]
    mesh = plsc.VectorSubcoreMesh(core_axis_name="core",
                                  subcore_axis_name="subcore")

    @pl.kernel(
        out_type=jax.ShapeDtypeStruct((n_idx, emb), table.dtype),
        mesh=mesh,
        compiler_params=pltpu.CompilerParams(use_tc_tiling_on_sc=False),
    )
    def k(table_hbm, idx_hbm, out_hbm):
        def body(idx_vmem, out_vmem):
            pltpu.sync_copy(table_hbm.at[idx_vmem.at[0]], out_vmem)

        pltpu.emit_pipeline(
            body,
            grid=(n_idx // _WINDOW,),
            in_specs=[pl.BlockSpec((1, _WINDOW), index_map=lambda i: (0, i))],
            out_specs=[pl.BlockSpec((_WINDOW, emb), index_map=lambda i: (i, 0))],
            core_axis_name=("core", "subcore"),
            dimension_semantics=(pltpu.PARALLEL,),
        )(idx_hbm, out_hbm)

    return k(table, tokens_flat.reshape(1, n_idx))


def kernel(tokens, table):
    batch, hist = tokens.shape
    flat = tokens.reshape(batch * hist)
    out = _embed_sc(flat, table)
    return out.reshape(batch, hist, table.shape[1])
